# baseline (speedup 1.0000x reference)
import functools

import jax
import jax.numpy as jnp
from jax import lax
from jax.experimental import pallas as pl
from jax.experimental.pallas import tpu as pltpu
from jax.experimental.pallas import tpu_sc as plsc

BATCH = 16384
CODE = 64
NSC = 2
BPC = BATCH // NSC
CHUNK = 1024

_mesh = plsc.ScalarSubcoreMesh(axis_name="c", num_cores=NSC)


@functools.partial(
    pl.kernel,
    mesh=_mesh,
    out_type=(
        jax.ShapeDtypeStruct((BATCH, CODE), jnp.float32),
        jax.ShapeDtypeStruct((BATCH, CODE), jnp.float32),
    ),
    scratch_types=[
        pltpu.SMEM((CHUNK,), jnp.int32),
        pltpu.SemaphoreType.DMA,
        pltpu.SemaphoreType.DMA,
        pltpu.SemaphoreType.DMA,
    ],
)
def _gather2(idx_hbm, ws_hbm, wa_hbm, out_s, out_a, idx_sm, sem_s, sem_a, sem_i):
    cid = lax.axis_index("c")
    base = cid * BPC

    def chunk_body(ch, carry):
        off = base + ch * CHUNK
        pltpu.make_async_copy(idx_hbm.at[pl.ds(off, CHUNK)], idx_sm, sem_i).start()
        pltpu.make_async_copy(idx_hbm.at[pl.ds(off, CHUNK)], idx_sm, sem_i).wait()

        def body(i, c):
            r = idx_sm[i]
            pltpu.make_async_copy(
                ws_hbm.at[pl.ds(r, 1)], out_s.at[pl.ds(off + i, 1)], sem_s).start()
            pltpu.make_async_copy(
                wa_hbm.at[pl.ds(r, 1)], out_a.at[pl.ds(off + i, 1)], sem_a).start()
            return c

        return lax.fori_loop(0, CHUNK, body, carry, unroll=8)

    lax.fori_loop(0, BPC // CHUNK, chunk_body, 0)
    pltpu.make_async_copy(
        ws_hbm.at[pl.ds(0, BPC)], out_s.at[pl.ds(base, BPC)], sem_s).wait()
    pltpu.make_async_copy(
        wa_hbm.at[pl.ds(0, BPC)], out_a.at[pl.ds(base, BPC)], sem_a).wait()


def kernel(instance_ids, W_shape, W_appearance):
    idx = instance_ids.astype(jnp.int32)
    return _gather2(idx, W_shape, W_appearance)


# speedup vs baseline: 3.5074x; 3.5074x over previous
import functools

import jax
import jax.numpy as jnp
from jax import lax
from jax.experimental import pallas as pl
from jax.experimental.pallas import tpu as pltpu
from jax.experimental.pallas import tpu_sc as plsc

BATCH = 16384
CODE = 64
NW = 32
EPW = BATCH // NW
RING = 32

_mesh = plsc.VectorSubcoreMesh(core_axis_name="c", subcore_axis_name="s")
_params = pltpu.CompilerParams(needs_layout_passes=False)


def _bcast(x):
    return jnp.full((16,), 0, jnp.int32) + x


def _vread(ref, i):
    return plsc.load_gather(ref, [_bcast(i)])[0]


@functools.partial(
    pl.kernel,
    mesh=_mesh,
    out_type=(
        jax.ShapeDtypeStruct((BATCH * CODE,), jnp.float32),
        jax.ShapeDtypeStruct((BATCH * CODE,), jnp.float32),
    ),
    scratch_types=[
        pltpu.VMEM((EPW,), jnp.int32),
        pltpu.VMEM((EPW,), jnp.int32),
        pltpu.VMEM((EPW + 32,), jnp.int32),
        pltpu.VMEM((EPW + 32,), jnp.int32),
        pltpu.VMEM((2 * CODE, 128), jnp.float32),
        pltpu.VMEM((2 * CODE, 128), jnp.float32),
        pltpu.VMEM((RING * CODE,), jnp.float32),
        pltpu.VMEM((RING * CODE,), jnp.float32),
        pltpu.SemaphoreType.DMA,
        pltpu.SemaphoreType.DMA,
        pltpu.SemaphoreType.DMA,
    ],
    compiler_params=_params,
)
def _k1(sidx_hbm, pos_hbm, ws_hbm, wa_hbm, stage_s, stage_a,
        sidx, pos, blk_m1, dstart, slab_s, slab_a, col_s, col_a,
        sem_f0, sem_f1, sem_w):
    wid = lax.axis_index("s") * 2 + lax.axis_index("c")
    base = wid * EPW
    pltpu.sync_copy(sidx_hbm.at[pl.ds(base, EPW)], sidx)
    pltpu.sync_copy(pos_hbm.at[pl.ds(base, EPW)], pos)

    blk_m1[pl.ds(0, 16)] = _bcast(-1)

    def blocks_body(c, carry):
        v = sidx[pl.ds(c * 16, 16)]
        blk_m1[pl.ds(1 + c * 16, 16)] = lax.shift_right_logical(v, 7)
        return carry

    lax.fori_loop(0, EPW // 16, blocks_body, 0)

    def dstart_body(i, wptr):
        is_new = _vread(blk_m1, 1 + i) != _vread(blk_m1, i)

        def write():
            dstart[pl.ds(wptr, 16)] = _bcast(i)
            return wptr + 1

        return lax.cond(is_new, write, lambda: wptr)

    cnt = lax.fori_loop(0, EPW, dstart_body, 0)
    dstart[pl.ds(cnt, 16)] = _bcast(EPW)

    def fetch(d, slot, sem):
        i0 = _vread(dstart, d)
        b = lax.shift_right_logical(_vread(sidx, i0), 7)
        off = pl.multiple_of(b * 128, 128)
        pltpu.async_copy(ws_hbm.at[:, pl.ds(off, 128)],
                         slab_s.at[pl.ds(slot * CODE, CODE)], sem)
        pltpu.async_copy(wa_hbm.at[:, pl.ds(off, 128)],
                         slab_a.at[pl.ds(slot * CODE, CODE)], sem)

    def wait_pair(sem):
        pltpu.make_async_copy(ws_hbm.at[:, pl.ds(0, 128)],
                              slab_s.at[pl.ds(0, CODE)], sem).wait()
        pltpu.make_async_copy(wa_hbm.at[:, pl.ds(0, 128)],
                              slab_a.at[pl.ds(0, CODE)], sem).wait()

    def extract_block(d, slot, e0):
        rs = _vread(dstart, d)
        re = _vread(dstart, d + 1)

        def entry_body(i, e):
            lane = _bcast(_vread(sidx, i) & 127)
            p = _vread(pos, i)
            ring = (e % RING) * CODE
            for cc in range(CODE // 16):
                rows = lax.iota(jnp.int32, 16) + (slot * CODE + cc * 16)
                col_s[pl.ds(ring + cc * 16, 16)] = plsc.load_gather(
                    slab_s, [rows, lane])
                col_a[pl.ds(ring + cc * 16, 16)] = plsc.load_gather(
                    slab_a, [rows, lane])

            @pl.when(e >= RING)
            def _():
                pltpu.make_async_copy(col_s.at[pl.ds(0, CODE)],
                                      stage_s.at[pl.ds(0, CODE)], sem_w).wait()
                pltpu.make_async_copy(col_a.at[pl.ds(0, CODE)],
                                      stage_a.at[pl.ds(0, CODE)], sem_w).wait()

            pltpu.async_copy(col_s.at[pl.ds(ring, CODE)],
                             stage_s.at[pl.ds(p * CODE, CODE)], sem_w)
            pltpu.async_copy(col_a.at[pl.ds(ring, CODE)],
                             stage_a.at[pl.ds(p * CODE, CODE)], sem_w)
            return e + 1

        return lax.fori_loop(rs, re, entry_body, e0)

    fetch(0, 0, sem_f0)

    def pingpong(dd, e):
        d0 = 2 * dd

        @pl.when(d0 + 1 < cnt)
        def _():
            fetch(d0 + 1, 1, sem_f1)

        def half0(e):
            wait_pair(sem_f0)
            return extract_block(d0, 0, e)

        e = lax.cond(d0 < cnt, half0, lambda v: v, e)

        @pl.when(d0 + 2 < cnt)
        def _():
            fetch(d0 + 2, 0, sem_f0)

        def half1(e):
            wait_pair(sem_f1)
            return extract_block(d0 + 1, 1, e)

        return lax.cond(d0 + 1 < cnt, half1, lambda v: v, e)

    lax.fori_loop(0, EPW // 2, pingpong, 0)

    pltpu.make_async_copy(col_s.at[pl.ds(0, RING * CODE)],
                          stage_s.at[pl.ds(0, RING * CODE)], sem_w).wait()
    pltpu.make_async_copy(col_a.at[pl.ds(0, RING * CODE)],
                          stage_a.at[pl.ds(0, RING * CODE)], sem_w).wait()


@functools.partial(
    pl.kernel,
    mesh=_mesh,
    out_type=(
        jax.ShapeDtypeStruct((CODE, BATCH), jnp.float32),
        jax.ShapeDtypeStruct((CODE, BATCH), jnp.float32),
    ),
    scratch_types=[
        pltpu.VMEM((EPW * CODE,), jnp.float32),
        pltpu.VMEM((CODE, EPW), jnp.float32),
    ],
    compiler_params=_params,
)
def _k2(stage_s, stage_a, out_s, out_a, lin, blk):
    wid = lax.axis_index("s") * 2 + lax.axis_index("c")
    base = wid * EPW

    for stage, out in ((stage_s, out_s), (stage_a, out_a)):
        pltpu.sync_copy(stage.at[pl.ds(base * CODE, EPW * CODE)], lin)

        def jc_body(jc, carry):
            jbase = jc * 16
            for c in range(CODE):
                idxv = (lax.iota(jnp.int32, 16) + jbase) * CODE + c
                blk[c, pl.ds(jbase, 16)] = plsc.load_gather(lin, [idxv])
            return carry

        lax.fori_loop(0, EPW // 16, jc_body, 0)
        pltpu.sync_copy(blk, out.at[:, pl.ds(base, EPW)])


def kernel(instance_ids, W_shape, W_appearance):
    idx = instance_ids.astype(jnp.int32)
    order = jnp.argsort(idx).astype(jnp.int32)
    sidx = jnp.take(idx, order)
    stage_s, stage_a = _k1(sidx, order, W_shape.T, W_appearance.T)
    out_s_t, out_a_t = _k2(stage_s, stage_a)
    return out_s_t.T, out_a_t.T


# speedup vs baseline: 3.6032x; 1.0273x over previous
import functools

import jax
import jax.numpy as jnp
from jax import lax
from jax.experimental import pallas as pl
from jax.experimental.pallas import tpu as pltpu
from jax.experimental.pallas import tpu_sc as plsc

BATCH = 16384
CODE = 64
NW = 32
EPW = BATCH // NW
RING = 32

_mesh = plsc.VectorSubcoreMesh(core_axis_name="c", subcore_axis_name="s")
_params = pltpu.CompilerParams(needs_layout_passes=False)


def _bcast(x):
    return jnp.full((16,), 0, jnp.int32) + x


def _vread(ref, i):
    return plsc.load_gather(ref, [_bcast(i)])[0]


@functools.partial(
    pl.kernel,
    mesh=_mesh,
    out_type=(
        jax.ShapeDtypeStruct((BATCH * CODE,), jnp.float32),
        jax.ShapeDtypeStruct((BATCH * CODE,), jnp.float32),
    ),
    scratch_types=[
        pltpu.VMEM((EPW,), jnp.int32),
        pltpu.VMEM((EPW,), jnp.int32),
        pltpu.VMEM((EPW + 32,), jnp.int32),
        pltpu.VMEM((EPW + 32,), jnp.int32),
        pltpu.VMEM((2 * CODE, 128), jnp.float32),
        pltpu.VMEM((2 * CODE, 128), jnp.float32),
        pltpu.VMEM((RING * CODE,), jnp.float32),
        pltpu.VMEM((RING * CODE,), jnp.float32),
        pltpu.SemaphoreType.DMA,
        pltpu.SemaphoreType.DMA,
        pltpu.SemaphoreType.DMA,
    ],
    compiler_params=_params,
)
def _k1(sidx_hbm, pos_hbm, ws_hbm, wa_hbm, stage_s, stage_a,
        sidx, pos, blk_m1, dstart, slab_s, slab_a, col_s, col_a,
        sem_f0, sem_f1, sem_w):
    wid = lax.axis_index("s") * 2 + lax.axis_index("c")
    base = wid * EPW
    pltpu.sync_copy(sidx_hbm.at[pl.ds(base, EPW)], sidx)
    pltpu.sync_copy(pos_hbm.at[pl.ds(base, EPW)], pos)

    blk_m1[pl.ds(0, 16)] = _bcast(-1)

    def blocks_body(c, carry):
        v = sidx[pl.ds(c * 16, 16)]
        blk_m1[pl.ds(1 + c * 16, 16)] = lax.shift_right_logical(v, 7)
        return carry

    lax.fori_loop(0, EPW // 16, blocks_body, 0)

    def dstart_body(i, wptr):
        is_new = _vread(blk_m1, 1 + i) != _vread(blk_m1, i)

        def write():
            dstart[pl.ds(wptr, 16)] = _bcast(i)
            return wptr + 1

        return lax.cond(is_new, write, lambda: wptr)

    cnt = lax.fori_loop(0, EPW, dstart_body, 0)
    dstart[pl.ds(cnt, 16)] = _bcast(EPW)

    def fetch(d, slot, sem):
        i0 = _vread(dstart, d)
        b = lax.shift_right_logical(_vread(sidx, i0), 7)
        off = pl.multiple_of(b * 128, 128)
        pltpu.async_copy(ws_hbm.at[:, pl.ds(off, 128)],
                         slab_s.at[pl.ds(slot * CODE, CODE)], sem)
        pltpu.async_copy(wa_hbm.at[:, pl.ds(off, 128)],
                         slab_a.at[pl.ds(slot * CODE, CODE)], sem)

    def wait_pair(sem):
        pltpu.make_async_copy(ws_hbm.at[:, pl.ds(0, 128)],
                              slab_s.at[pl.ds(0, CODE)], sem).wait()
        pltpu.make_async_copy(wa_hbm.at[:, pl.ds(0, 128)],
                              slab_a.at[pl.ds(0, CODE)], sem).wait()

    def extract_block(d, slot, e0):
        rs = _vread(dstart, d)
        re = _vread(dstart, d + 1)

        def entry_body(i, e):
            lane = _bcast(_vread(sidx, i) & 127)
            p = _vread(pos, i)
            ring = (e % RING) * CODE
            for cc in range(CODE // 16):
                rows = lax.iota(jnp.int32, 16) + (slot * CODE + cc * 16)
                col_s[pl.ds(ring + cc * 16, 16)] = plsc.load_gather(
                    slab_s, [rows, lane])
                col_a[pl.ds(ring + cc * 16, 16)] = plsc.load_gather(
                    slab_a, [rows, lane])

            @pl.when(e >= RING)
            def _():
                pltpu.make_async_copy(col_s.at[pl.ds(0, CODE)],
                                      stage_s.at[pl.ds(0, CODE)], sem_w).wait()
                pltpu.make_async_copy(col_a.at[pl.ds(0, CODE)],
                                      stage_a.at[pl.ds(0, CODE)], sem_w).wait()

            pltpu.async_copy(col_s.at[pl.ds(ring, CODE)],
                             stage_s.at[pl.ds(p * CODE, CODE)], sem_w)
            pltpu.async_copy(col_a.at[pl.ds(ring, CODE)],
                             stage_a.at[pl.ds(p * CODE, CODE)], sem_w)
            return e + 1

        return lax.fori_loop(rs, re, entry_body, e0)

    fetch(0, 0, sem_f0)

    def pingpong(dd, e):
        d0 = 2 * dd

        @pl.when(d0 + 1 < cnt)
        def _():
            fetch(d0 + 1, 1, sem_f1)

        def half0(e):
            wait_pair(sem_f0)
            return extract_block(d0, 0, e)

        e = lax.cond(d0 < cnt, half0, lambda v: v, e)

        @pl.when(d0 + 2 < cnt)
        def _():
            fetch(d0 + 2, 0, sem_f0)

        def half1(e):
            wait_pair(sem_f1)
            return extract_block(d0 + 1, 1, e)

        return lax.cond(d0 + 1 < cnt, half1, lambda v: v, e)

    lax.fori_loop(0, EPW // 2, pingpong, 0)

    pltpu.make_async_copy(col_s.at[pl.ds(0, RING * CODE)],
                          stage_s.at[pl.ds(0, RING * CODE)], sem_w).wait()
    pltpu.make_async_copy(col_a.at[pl.ds(0, RING * CODE)],
                          stage_a.at[pl.ds(0, RING * CODE)], sem_w).wait()


@functools.partial(
    pl.kernel,
    mesh=_mesh,
    out_type=(
        jax.ShapeDtypeStruct((CODE, BATCH), jnp.float32),
        jax.ShapeDtypeStruct((CODE, BATCH), jnp.float32),
    ),
    scratch_types=[
        pltpu.VMEM((EPW * CODE,), jnp.float32),
        pltpu.VMEM((EPW * CODE,), jnp.float32),
        pltpu.VMEM((CODE, EPW), jnp.float32),
        pltpu.SemaphoreType.DMA,
        pltpu.SemaphoreType.DMA,
    ],
    compiler_params=_params,
)
def _k2(stage_s, stage_a, out_s, out_a, lin_s, lin_a, blk, sem_s, sem_a):
    wid = lax.axis_index("s") * 2 + lax.axis_index("c")
    base = wid * EPW
    pltpu.async_copy(stage_s.at[pl.ds(base * CODE, EPW * CODE)], lin_s, sem_s)
    pltpu.async_copy(stage_a.at[pl.ds(base * CODE, EPW * CODE)], lin_a, sem_a)

    for stage, lin, sem, out in ((stage_s, lin_s, sem_s, out_s),
                                 (stage_a, lin_a, sem_a, out_a)):
        pltpu.make_async_copy(
            stage.at[pl.ds(0, EPW * CODE)], lin, sem).wait()

        def jc_body(jc, carry, lin=lin):
            jbase = jc * 16
            for c in range(CODE):
                idxv = (lax.iota(jnp.int32, 16) + jbase) * CODE + c
                blk[c, pl.ds(jbase, 16)] = plsc.load_gather(lin, [idxv])
            return carry

        lax.fori_loop(0, EPW // 16, jc_body, 0)
        pltpu.sync_copy(blk, out.at[:, pl.ds(base, EPW)])


def kernel(instance_ids, W_shape, W_appearance):
    idx = instance_ids.astype(jnp.int32)
    sidx, order = lax.sort(
        (idx, lax.iota(jnp.int32, BATCH)), num_keys=1)
    stage_s, stage_a = _k1(sidx, order, W_shape.T, W_appearance.T)
    out_s_t, out_a_t = _k2(stage_s, stage_a)
    return out_s_t.T, out_a_t.T
